# baseline (speedup 1.0000x reference)
import functools

import jax
import jax.numpy as jnp
from jax import lax
from jax.experimental import pallas as pl
from jax.experimental.pallas import tpu as pltpu
from jax.experimental.pallas import tpu_sc as plsc

N = 10000
E = 320000
DH = 256
HH = 128
D_E = 16
NEG_SLOPE = 0.2

NC = 2
NS = 16
NW = NC * NS
E_W_A = E // NW
E_W_D = E // NS
CA = 80
CC = 2000
CD = 80
NP_ = 10240
NSL = NP_ // NS

F32 = jnp.float32


@functools.lru_cache(maxsize=None)
def _mesh():
    return plsc.VectorSubcoreMesh(core_axis_name="c", subcore_axis_name="s",
                                  num_cores=NC, num_subcores=NS)


def _allsum16(v):
    lanes = lax.iota(jnp.int32, 16)
    for d in (1, 2, 4, 8):
        v = v + jnp.take_along_axis(v, lanes ^ d, axis=0,
                                    mode="promise_in_bounds")
    return v


def _seg_reduce16(keys, vals, op):
    sk, sv = plsc.sort_key_val(keys, vals)
    lanes = lax.iota(jnp.int32, 16)
    for d in (1, 2, 4, 8):
        idx = jnp.maximum(lanes - d, 0)
        pk = jnp.take_along_axis(sk, idx, axis=0, mode="promise_in_bounds")
        pv = jnp.take_along_axis(sv, idx, axis=0, mode="promise_in_bounds")
        ok = (lanes >= d) & (pk == sk)
        sv = jnp.where(ok, op(sv, pv), sv)
    nidx = jnp.minimum(lanes + 1, 15)
    nk = jnp.take_along_axis(sk, nidx, axis=0, mode="promise_in_bounds")
    is_last = (lanes == 15) | (nk != sk)
    return sk, sv, is_last


def _table_update(tab, keys, vals, op):
    sk, sv, is_last = _seg_reduce16(keys, vals, op)
    cur = plsc.load_gather(tab, [sk])
    plsc.store_scatter(tab, [sk], op(cur, sv), mask=is_last)


def _vloop(ref, nwords, fn):
    def body(i, _):
        ref[pl.ds(i * 16, 16)] = fn(i)
        return _
    lax.fori_loop(0, nwords // 16, body, None)


ABLK = 25
NCH_A = E_W_A // CA


def _k_attn(xlb, xrb, ewb, src, dst, att,
            logits, mpart,
            m_t, ba0, bb0, bc0, ba1, bb1, bc1,
            bsrc, bdst, blog, batt, tmpa, tmpb, m_sh, sg0, sg1):
    cid = lax.axis_index("c")
    sid = lax.axis_index("s")
    wid = sid * NC + cid

    neg = jnp.full((16,), -jnp.inf, F32)
    _vloop(m_t, NP_, lambda i: neg)
    pltpu.sync_copy(att, batt)

    ebase = wid * E_W_A
    gb = ((ba0, bb0, bc0), (ba1, bb1, bc1))
    sems = (sg0, sg1)

    def start_gathers(p, ebb, j):
        a, b, c = gb[p]
        s = sems[p]
        si = bsrc.at[pl.ds(j * CA, CA)]
        di = bdst.at[pl.ds(j * CA, CA)]
        pltpu.async_copy(xlb.at[si], a, s)
        pltpu.async_copy(xrb.at[di], b, s)
        pltpu.async_copy(ewb.at[pl.ds(ebb + j * CA, CA)], c, s)

    def wait_gathers(p):
        a, b, c = gb[p]
        s = sems[p]
        i0 = bsrc.at[pl.ds(0, CA)]
        pltpu.make_async_copy(xlb.at[i0], a, s).wait()
        pltpu.make_async_copy(xrb.at[i0], b, s).wait()
        pltpu.make_async_copy(ewb.at[pl.ds(0, CA)], c, s).wait()

    lanes = lax.iota(jnp.int32, 16)

    def compute(p, ebb, j):
        a, b, c = gb[p]
        for g in range(CA // 16):
            def edge(jj, logv):
                e = g * 16 + jj
                acc = jnp.zeros((16,), F32)
                for cc in range(8):
                    xl2 = plsc.unpack(
                        plsc.bitcast(a[e, pl.ds(cc * 16, 16)], jnp.bfloat16),
                        format=plsc.PackFormat.INTERLEAVED)
                    xr2 = plsc.unpack(
                        plsc.bitcast(b[e, pl.ds(cc * 16, 16)], jnp.bfloat16),
                        format=plsc.PackFormat.INTERLEAVED)
                    ew2 = plsc.unpack(
                        plsc.bitcast(c[e, pl.ds(cc * 16, 16)], jnp.bfloat16),
                        format=plsc.PackFormat.INTERLEAVED)
                    for h in range(2):
                        s1 = xl2[h] + xr2[h] + ew2[h]
                        l1 = jnp.maximum(s1, s1 * NEG_SLOPE)
                        acc = acc + l1 * batt[2 * cc + h]
                return jnp.where(lanes == jj, _allsum16(acc), logv)
            logv = lax.fori_loop(0, 16, edge, jnp.zeros((16,), F32))
            blog[pl.ds(g * 16, 16)] = logv
            k16 = bdst[pl.ds(j * CA + g * 16, 16)]
            _table_update(m_t, k16, logv, jnp.maximum)
        pltpu.sync_copy(blog, logits.at[pl.ds(ebb + j * CA, CA)])

    def block(bb, carry):
        ebb = ebase + bb * (ABLK * CA)
        pltpu.sync_copy(src.at[pl.ds(ebb, ABLK * CA)], bsrc)
        pltpu.sync_copy(dst.at[pl.ds(ebb, ABLK * CA)], bdst)
        start_gathers(0, ebb, 0)

        def step(j, c2):
            def proc(p):
                @pl.when(j + 1 < ABLK)
                def _():
                    start_gathers(1 - p, ebb, j + 1)
                wait_gathers(p)
                compute(p, ebb, j)

            @pl.when(j % 2 == 0)
            def _():
                proc(0)

            @pl.when(j % 2 == 1)
            def _():
                proc(1)
            return c2
        lax.fori_loop(0, ABLK, step, None)
        return carry
    lax.fori_loop(0, NCH_A // ABLK, block, None)

    pltpu.sync_copy(m_t, m_sh.at[sid])
    plsc.subcore_barrier()
    nb = sid * NSL
    pltpu.sync_copy(m_sh.at[0, pl.ds(nb, NSL)], tmpa)
    for w in range(1, NS):
        pltpu.sync_copy(m_sh.at[w, pl.ds(nb, NSL)], tmpb)
        _vloop(tmpa, NSL,
               lambda i: jnp.maximum(tmpa[pl.ds(i * 16, 16)],
                                     tmpb[pl.ds(i * 16, 16)]))
    pltpu.sync_copy(tmpa, mpart.at[cid, pl.ds(nb, NSL)])


@functools.lru_cache(maxsize=None)
def _attn_call():
  return pl.kernel(
    _k_attn,
    out_type=(jax.ShapeDtypeStruct((E,), F32),
              jax.ShapeDtypeStruct((NC, NP_), F32)),
    mesh=_mesh(),
    compiler_params=pltpu.CompilerParams(needs_layout_passes=False),
    scratch_types=[
        pltpu.VMEM((NP_,), F32),
        pltpu.VMEM((CA, HH), jnp.int32),
        pltpu.VMEM((CA, HH), jnp.int32),
        pltpu.VMEM((CA, HH), jnp.int32),
        pltpu.VMEM((CA, HH), jnp.int32),
        pltpu.VMEM((CA, HH), jnp.int32),
        pltpu.VMEM((CA, HH), jnp.int32),
        pltpu.VMEM((ABLK * CA,), jnp.int32),
        pltpu.VMEM((ABLK * CA,), jnp.int32),
        pltpu.VMEM((CA,), F32),
        pltpu.VMEM((16, 16), F32),
        pltpu.VMEM((NSL,), F32),
        pltpu.VMEM((NSL,), F32),
        pltpu.VMEM_SHARED((NS, NP_), F32),
        pltpu.SemaphoreType.DMA,
        pltpu.SemaphoreType.DMA,
    ],
  )


def _k_softmax(logits, dst, mpart,
               ex, dpart,
               m_full, den_t, blog, bdst, bex, tmpa, tmpb, den_sh, sem):
    cid = lax.axis_index("c")
    sid = lax.axis_index("s")
    wid = sid * NC + cid

    pltpu.sync_copy(mpart.at[0], m_full)
    pltpu.sync_copy(mpart.at[1], den_t)
    _vloop(m_full, NP_,
           lambda i: jnp.maximum(m_full[pl.ds(i * 16, 16)],
                                 den_t[pl.ds(i * 16, 16)]))
    zero = jnp.zeros((16,), F32)
    _vloop(den_t, NP_, lambda i: zero)

    ebase = wid * E_W_A

    def chunk(it, carry):
        base = ebase + it * CC
        pltpu.sync_copy(logits.at[pl.ds(base, CC)], blog)
        pltpu.sync_copy(dst.at[pl.ds(base, CC)], bdst)

        def grp(g, c2):
            l16 = blog[pl.ds(g * 16, 16)]
            d16 = bdst[pl.ds(g * 16, 16)]
            mv = plsc.load_gather(m_full, [d16])
            e16 = jnp.exp(l16 - mv)
            bex[pl.ds(g * 16, 16)] = e16
            _table_update(den_t, d16, e16, lambda a, b: a + b)
            return c2
        lax.fori_loop(0, CC // 16, grp, None)
        pltpu.sync_copy(bex, ex.at[pl.ds(base, CC)])
        return carry
    lax.fori_loop(0, E_W_A // CC, chunk, None)

    pltpu.sync_copy(den_t, den_sh.at[sid])
    plsc.subcore_barrier()
    nb = sid * NSL
    pltpu.sync_copy(den_sh.at[0, pl.ds(nb, NSL)], tmpa)
    for w in range(1, NS):
        pltpu.sync_copy(den_sh.at[w, pl.ds(nb, NSL)], tmpb)
        _vloop(tmpa, NSL,
               lambda i: tmpa[pl.ds(i * 16, 16)] + tmpb[pl.ds(i * 16, 16)])
    pltpu.sync_copy(tmpa, dpart.at[cid, pl.ds(nb, NSL)])


@functools.lru_cache(maxsize=None)
def _softmax_call():
  return pl.kernel(
    _k_softmax,
    out_type=(jax.ShapeDtypeStruct((E,), F32),
              jax.ShapeDtypeStruct((NC, NP_), F32)),
    mesh=_mesh(),
    compiler_params=pltpu.CompilerParams(needs_layout_passes=False),
    scratch_types=[
        pltpu.VMEM((NP_,), F32),
        pltpu.VMEM((NP_,), F32),
        pltpu.VMEM((CC,), F32),
        pltpu.VMEM((CC,), jnp.int32),
        pltpu.VMEM((CC,), F32),
        pltpu.VMEM((NSL,), F32),
        pltpu.VMEM((NSL,), F32),
        pltpu.VMEM_SHARED((NS, NP_), F32),
        pltpu.SemaphoreType.DMA,
    ],
  )


NCH_D = E_W_D // CD
DBLK = 10
NPD = 10112
NSD = NPD // NS


def _k_aggr(ex, srcr, dstr, dpart, xllo, xlhi, biasr,
            hlo, hhi,
            den_full, brow0, brow1, bsrc, bdst, bex, bdsts, bbias,
            fbuf, out_sh, sg0, sg1, ss0, ss1, *, use_silu):
    cid = lax.axis_index("c")
    sid = lax.axis_index("s")

    pltpu.sync_copy(dpart.at[pl.ds(0, NP_)], den_full)
    for k, sz in [(kk, 800) for kk in range(0, 9600, 800)] + [(9600, 640)]:
        pltpu.sync_copy(dpart.at[pl.ds(NP_ + k, sz)], bex.at[pl.ds(0, sz)])

        def comb(i, c2, k=k):
            den_full[pl.ds(k + i * 16, 16)] = (
                den_full[pl.ds(k + i * 16, 16)] + bex[pl.ds(i * 16, 16)])
            return c2
        lax.fori_loop(0, sz // 16, comb, None)
    pltpu.sync_copy(biasr.at[cid], bbias)

    ebase = sid * E_W_D

    zero = jnp.zeros((16,), F32)

    def zb(i, c2):
        r = i // 8
        cc = i % 8
        fbuf[r, pl.ds(cc * 16, 16)] = zero
        return c2
    lax.fori_loop(0, 8 * 8, zb, None)

    def core_half(xl_half, h_half):
        for b in range(NSD // 8):
            pltpu.sync_copy(fbuf, out_sh.at[pl.ds(sid * NSD + b * 8, 8)])
        plsc.subcore_barrier()

        brows = (brow0, brow1)
        sems = (sg0, sg1)
        ssem = (ss0, ss1)

        def start_g(p, j):
            pltpu.async_copy(xl_half.at[bsrc.at[pl.ds(j * CD, CD)]],
                             brows[p], sems[p])

        def wait_scatter(p):
            pltpu.make_async_copy(brows[p], out_sh.at[bdsts], ssem[p]).wait()

        def block(bb, c1):
            eblk = ebase + bb * (DBLK * CD)
            pltpu.sync_copy(srcr.at[pl.ds(eblk, DBLK * CD)], bsrc)
            pltpu.sync_copy(dstr.at[pl.ds(eblk, DBLK * CD)], bdst)
            pltpu.sync_copy(ex.at[pl.ds(eblk, DBLK * CD)], bex)
            start_g(0, 0)

            def step(j, c2):
                def proc(p):
                    cur = brows[p]

                    @pl.when((j > 0) | (bb > 0))
                    def _():
                        wait_scatter(1 - p)

                    @pl.when(j + 1 < DBLK)
                    def _():
                        start_g(1 - p, j + 1)
                    pltpu.make_async_copy(
                        xl_half.at[bsrc.at[pl.ds(0, CD)]], cur,
                        sems[p]).wait()

                    def grp(g, c3):
                        d16 = bdst[pl.ds(j * CD + g * 16, 16)]
                        bdsts[pl.ds(g * 16, 16)] = d16
                        e16 = bex[pl.ds(j * CD + g * 16, 16)]
                        dv = plsc.load_gather(den_full, [d16])
                        av16 = e16 / (dv + 1e-16)
                        for jj in range(16):
                            avj = jnp.take_along_axis(
                                av16, jnp.full((16,), jj, jnp.int32),
                                axis=0, mode="promise_in_bounds")
                            e2 = g * 16 + jj
                            for cc in range(8):
                                cur[e2, pl.ds(cc * 16, 16)] = (
                                    cur[e2, pl.ds(cc * 16, 16)] * avj)
                        return c3
                    lax.fori_loop(0, CD // 16, grp, None)
                    pltpu.async_copy(cur, out_sh.at[bdsts], ssem[p],
                                     add=True)

                @pl.when(j % 2 == 0)
                def _():
                    proc(0)

                @pl.when(j % 2 == 1)
                def _():
                    proc(1)
                return c2
            lax.fori_loop(0, DBLK, step, None)
            return c1
        lax.fori_loop(0, NCH_D // DBLK, block, None)
        wait_scatter((DBLK - 1) % 2)
        plsc.subcore_barrier()

        def finblk(b, c1):
            rbase = sid * NSD + b * 8
            pltpu.sync_copy(out_sh.at[pl.ds(rbase, 8)], fbuf)

            def fin(i, c2):
                r = i // 8
                cc = i % 8
                v = fbuf[r, pl.ds(cc * 16, 16)] + bbias[cc]
                if use_silu:
                    v = v / (1.0 + jnp.exp(-v))
                fbuf[r, pl.ds(cc * 16, 16)] = v
                return c2
            lax.fori_loop(0, 8 * 8, fin, None)
            pltpu.sync_copy(fbuf, h_half.at[pl.ds(rbase, 8)])
            return c1
        lax.fori_loop(0, NSD // 8, finblk, None)

    @pl.when(cid == 0)
    def _():
        core_half(xllo, hlo)

    @pl.when(cid == 1)
    def _():
        core_half(xlhi, hhi)


@functools.lru_cache(maxsize=None)
def _make_aggr(use_silu):
    return pl.kernel(
        functools.partial(_k_aggr, use_silu=use_silu),
        out_type=(jax.ShapeDtypeStruct((NPD, HH), F32),
                  jax.ShapeDtypeStruct((NPD, HH), F32)),
        mesh=_mesh(),
        compiler_params=pltpu.CompilerParams(needs_layout_passes=False),
        scratch_types=[
            pltpu.VMEM((NP_,), F32),
            pltpu.VMEM((CD, HH), F32),
            pltpu.VMEM((CD, HH), F32),
            pltpu.VMEM((DBLK * CD,), jnp.int32),
            pltpu.VMEM((DBLK * CD,), jnp.int32),
            pltpu.VMEM((DBLK * CD,), F32),
            pltpu.VMEM((CD,), jnp.int32),
            pltpu.VMEM((8, 16), F32),
            pltpu.VMEM((8, HH), F32),
            pltpu.VMEM_SHARED((NPD, HH), F32),
            pltpu.SemaphoreType.DMA,
            pltpu.SemaphoreType.DMA,
            pltpu.SemaphoreType.DMA,
            pltpu.SemaphoreType.DMA,
        ],
    )


def _pack_pair(lo, hi):
    bl = lax.bitcast_convert_type(lo, jnp.uint32)
    bh = lax.bitcast_convert_type(hi, jnp.uint32)
    rl = (bl + 0x7FFF + ((bl >> 16) & 1)) >> 16
    rh = ((bh + 0x7FFF + ((bh >> 16) & 1)) >> 16) << 16
    return lax.bitcast_convert_type(rl | rh, jnp.int32)


def _pack_words(xf):
    return _pack_pair(xf[:, :HH], xf[:, HH:])


def _proj1_body(x_ref, wl_ref, bl_ref, wr_ref, br_ref,
                xllo, xlhi, xlb, xrb):
    xv = x_ref[...]
    xl = jnp.dot(xv, wl_ref[...], preferred_element_type=F32,
                 precision=lax.Precision.HIGHEST) + bl_ref[...]
    xr = jnp.dot(xv, wr_ref[...], preferred_element_type=F32,
                 precision=lax.Precision.HIGHEST) + br_ref[...]
    xllo[...] = xl[:, :HH]
    xlhi[...] = xl[:, HH:]
    xlb[...] = _pack_words(xl)
    xrb[...] = _pack_words(xr)


_RB = 1000

_proj1 = pl.pallas_call(
    _proj1_body,
    grid=(N // _RB,),
    in_specs=[
        pl.BlockSpec((_RB, 128), lambda i: (i, 0)),
        pl.BlockSpec((128, DH), lambda i: (0, 0)),
        pl.BlockSpec((DH,), lambda i: (0,)),
        pl.BlockSpec((128, DH), lambda i: (0, 0)),
        pl.BlockSpec((DH,), lambda i: (0,)),
    ],
    out_specs=[pl.BlockSpec((_RB, HH), lambda i: (i, 0))] * 4,
    out_shape=[jax.ShapeDtypeStruct((N, HH), F32)] * 2
    + [jax.ShapeDtypeStruct((N, HH), jnp.int32)] * 2,
)


def _proj2_body(xlo_ref, xhi_ref, wl_ref, bl_ref, wr_ref, br_ref,
                xllo, xlhi, xlb, xrb):
    lo = xlo_ref[...]
    hi = xhi_ref[...]
    xl = (jnp.dot(lo, wl_ref[:HH, :], preferred_element_type=F32,
                  precision=lax.Precision.HIGHEST)
          + jnp.dot(hi, wl_ref[HH:, :], preferred_element_type=F32,
                    precision=lax.Precision.HIGHEST) + bl_ref[...])
    xr = (jnp.dot(lo, wr_ref[:HH, :], preferred_element_type=F32,
                  precision=lax.Precision.HIGHEST)
          + jnp.dot(hi, wr_ref[HH:, :], preferred_element_type=F32,
                    precision=lax.Precision.HIGHEST) + br_ref[...])
    xllo[...] = xl[:, :HH]
    xlhi[...] = xl[:, HH:]
    xlb[...] = _pack_words(xl)
    xrb[...] = _pack_words(xr)


_proj2 = pl.pallas_call(
    _proj2_body,
    grid=(N // _RB,),
    in_specs=[
        pl.BlockSpec((_RB, HH), lambda i: (i, 0)),
        pl.BlockSpec((_RB, HH), lambda i: (i, 0)),
        pl.BlockSpec((DH, DH), lambda i: (0, 0)),
        pl.BlockSpec((DH,), lambda i: (0,)),
        pl.BlockSpec((DH, DH), lambda i: (0, 0)),
        pl.BlockSpec((DH,), lambda i: (0,)),
    ],
    out_specs=[pl.BlockSpec((_RB, HH), lambda i: (i, 0))] * 4,
    out_shape=[jax.ShapeDtypeStruct((N, HH), F32)] * 2
    + [jax.ShapeDtypeStruct((N, HH), jnp.int32)] * 2,
)


def _ew_body(ea_ref, we_ref, out_ref):
    out_ref[...] = _pack_words(
        jnp.dot(ea_ref[...], we_ref[...], preferred_element_type=F32,
                precision=lax.Precision.HIGHEST))


_EB = 4000

_ew_call = pl.pallas_call(
    _ew_body,
    grid=(E // _EB,),
    in_specs=[
        pl.BlockSpec((_EB, D_E), lambda i: (i, 0)),
        pl.BlockSpec((D_E, DH), lambda i: (0, 0)),
    ],
    out_specs=pl.BlockSpec((_EB, HH), lambda i: (i, 0)),
    out_shape=jax.ShapeDtypeStruct((E, HH), jnp.int32),
)


def _att_arrange(att):
    return att.reshape(2, 8, 16).transpose(1, 0, 2).reshape(16, 16)


def _gat_layer(xllo, xlhi, xlb, xrb, ewb, src, dst, att, bias, use_silu):
    att_r = _att_arrange(att)
    bias_r = jnp.stack([bias[:HH].reshape(8, 16), bias[HH:].reshape(8, 16)])
    logits, mpart = _attn_call()(xlb, xrb, ewb, src, dst, att_r)
    ex, dpart = _softmax_call()(logits, dst, mpart)
    call = _make_aggr(use_silu)
    hlo, hhi = call(ex, src, dst, dpart.reshape(2 * NP_), xllo, xlhi, bias_r)
    return hlo[:N], hhi[:N]


def kernel(x, edge_index, edge_attr,
           Wl0, bl0, Wr0, br0, We0, att0, bias0,
           Wl1, bl1, Wr1, br1, We1, att1, bias1):
    src = edge_index[0]
    dst = edge_index[1]
    ewb0 = _ew_call(edge_attr, We0)
    ewb1 = _ew_call(edge_attr, We1)
    xllo, xlhi, xlb, xrb = _proj1(x, Wl0, bl0, Wr0, br0)
    h0lo, h0hi = _gat_layer(xllo, xlhi, xlb, xrb, ewb0, src, dst,
                            att0, bias0, True)
    xllo1, xlhi1, xlb1, xrb1 = _proj2(h0lo, h0hi, Wl1, bl1, Wr1, br1)
    h1lo, h1hi = _gat_layer(xllo1, xlhi1, xlb1, xrb1, ewb1, src, dst,
                            att1, bias1, False)
    return jnp.concatenate([h1lo, h1hi], axis=1)


# speedup vs baseline: 1.0430x; 1.0430x over previous
import functools

import jax
import jax.numpy as jnp
from jax import lax
from jax.experimental import pallas as pl
from jax.experimental.pallas import tpu as pltpu
from jax.experimental.pallas import tpu_sc as plsc

N = 10000
E = 320000
DH = 256
HH = 128
D_E = 16
NEG_SLOPE = 0.2

NC = 2
NS = 16
NW = NC * NS
E_W_A = E // NW
E_W_D = E // NS
CA = 80
CC = 2000
CD = 80
NP_ = 10240
NSL = NP_ // NS

F32 = jnp.float32


@functools.lru_cache(maxsize=None)
def _mesh():
    return plsc.VectorSubcoreMesh(core_axis_name="c", subcore_axis_name="s",
                                  num_cores=NC, num_subcores=NS)


def _allsum16(v):
    lanes = lax.iota(jnp.int32, 16)
    for d in (1, 2, 4, 8):
        v = v + jnp.take_along_axis(v, lanes ^ d, axis=0,
                                    mode="promise_in_bounds")
    return v


def _seg_reduce16(keys, vals, op):
    sk, sv = plsc.sort_key_val(keys, vals)
    lanes = lax.iota(jnp.int32, 16)
    for d in (1, 2, 4, 8):
        idx = jnp.maximum(lanes - d, 0)
        pk = jnp.take_along_axis(sk, idx, axis=0, mode="promise_in_bounds")
        pv = jnp.take_along_axis(sv, idx, axis=0, mode="promise_in_bounds")
        ok = (lanes >= d) & (pk == sk)
        sv = jnp.where(ok, op(sv, pv), sv)
    nidx = jnp.minimum(lanes + 1, 15)
    nk = jnp.take_along_axis(sk, nidx, axis=0, mode="promise_in_bounds")
    is_last = (lanes == 15) | (nk != sk)
    return sk, sv, is_last


def _table_update(tab, keys, vals, op):
    sk, sv, is_last = _seg_reduce16(keys, vals, op)
    cur = plsc.load_gather(tab, [sk])
    plsc.store_scatter(tab, [sk], op(cur, sv), mask=is_last)


def _vloop(ref, nwords, fn):
    def body(i, _):
        ref[pl.ds(i * 16, 16)] = fn(i)
        return _
    lax.fori_loop(0, nwords // 16, body, None)


ABLK = 25
NCH_A = E_W_A // CA


def _k_attn(xlb, xrb, ewb, src, dst, att,
            logits, mpart,
            m_t, ba0, bb0, bc0, ba1, bb1, bc1,
            bsrc, bdst, blog, batt, tmpa, tmpb, m_sh, sg0, sg1):
    cid = lax.axis_index("c")
    sid = lax.axis_index("s")
    wid = sid * NC + cid

    neg = jnp.full((16,), -jnp.inf, F32)
    _vloop(m_t, NP_, lambda i: neg)
    pltpu.sync_copy(att, batt)

    ebase = wid * E_W_A
    gb = ((ba0, bb0, bc0), (ba1, bb1, bc1))
    sems = (sg0, sg1)

    def start_gathers(p, ebb, j):
        a, b, c = gb[p]
        s = sems[p]
        si = bsrc.at[pl.ds(j * CA, CA)]
        di = bdst.at[pl.ds(j * CA, CA)]
        pltpu.async_copy(xlb.at[si], a, s)
        pltpu.async_copy(xrb.at[di], b, s)
        pltpu.async_copy(ewb.at[pl.ds(ebb + j * CA, CA)], c, s)

    def wait_gathers(p):
        a, b, c = gb[p]
        s = sems[p]
        i0 = bsrc.at[pl.ds(0, CA)]
        pltpu.make_async_copy(xlb.at[i0], a, s).wait()
        pltpu.make_async_copy(xrb.at[i0], b, s).wait()
        pltpu.make_async_copy(ewb.at[pl.ds(0, CA)], c, s).wait()

    lanes = lax.iota(jnp.int32, 16)

    def compute(p, ebb, j):
        a, b, c = gb[p]
        for g in range(CA // 16):
            def edge(jj, logv):
                e = g * 16 + jj
                acc = jnp.zeros((16,), F32)
                for cc in range(8):
                    xl2 = plsc.unpack(
                        plsc.bitcast(a[e, pl.ds(cc * 16, 16)], jnp.bfloat16),
                        format=plsc.PackFormat.INTERLEAVED)
                    xr2 = plsc.unpack(
                        plsc.bitcast(b[e, pl.ds(cc * 16, 16)], jnp.bfloat16),
                        format=plsc.PackFormat.INTERLEAVED)
                    ew2 = plsc.unpack(
                        plsc.bitcast(c[e, pl.ds(cc * 16, 16)], jnp.bfloat16),
                        format=plsc.PackFormat.INTERLEAVED)
                    for h in range(2):
                        s1 = xl2[h] + xr2[h] + ew2[h]
                        l1 = jnp.maximum(s1, s1 * NEG_SLOPE)
                        acc = acc + l1 * batt[2 * cc + h]
                return jnp.where(lanes == jj, _allsum16(acc), logv)
            logv = lax.fori_loop(0, 16, edge, jnp.zeros((16,), F32))
            blog[pl.ds(g * 16, 16)] = logv
            k16 = bdst[pl.ds(j * CA + g * 16, 16)]
            _table_update(m_t, k16, logv, jnp.maximum)
        pltpu.sync_copy(blog, logits.at[pl.ds(ebb + j * CA, CA)])

    def block(bb, carry):
        ebb = ebase + bb * (ABLK * CA)
        pltpu.sync_copy(src.at[pl.ds(ebb, ABLK * CA)], bsrc)
        pltpu.sync_copy(dst.at[pl.ds(ebb, ABLK * CA)], bdst)
        start_gathers(0, ebb, 0)

        def step(j, c2):
            def proc(p):
                @pl.when(j + 1 < ABLK)
                def _():
                    start_gathers(1 - p, ebb, j + 1)
                wait_gathers(p)
                compute(p, ebb, j)

            @pl.when(j % 2 == 0)
            def _():
                proc(0)

            @pl.when(j % 2 == 1)
            def _():
                proc(1)
            return c2
        lax.fori_loop(0, ABLK, step, None)
        return carry
    lax.fori_loop(0, NCH_A // ABLK, block, None)

    pltpu.sync_copy(m_t, m_sh.at[sid])
    plsc.subcore_barrier()
    nb = sid * NSL
    pltpu.sync_copy(m_sh.at[0, pl.ds(nb, NSL)], tmpa)
    for w in range(1, NS):
        pltpu.sync_copy(m_sh.at[w, pl.ds(nb, NSL)], tmpb)
        _vloop(tmpa, NSL,
               lambda i: jnp.maximum(tmpa[pl.ds(i * 16, 16)],
                                     tmpb[pl.ds(i * 16, 16)]))
    pltpu.sync_copy(tmpa, mpart.at[cid, pl.ds(nb, NSL)])


@functools.lru_cache(maxsize=None)
def _attn_call():
  return pl.kernel(
    _k_attn,
    out_type=(jax.ShapeDtypeStruct((E,), F32),
              jax.ShapeDtypeStruct((NC, NP_), F32)),
    mesh=_mesh(),
    compiler_params=pltpu.CompilerParams(needs_layout_passes=False),
    scratch_types=[
        pltpu.VMEM((NP_,), F32),
        pltpu.VMEM((CA, HH), jnp.int32),
        pltpu.VMEM((CA, HH), jnp.int32),
        pltpu.VMEM((CA, HH), jnp.int32),
        pltpu.VMEM((CA, HH), jnp.int32),
        pltpu.VMEM((CA, HH), jnp.int32),
        pltpu.VMEM((CA, HH), jnp.int32),
        pltpu.VMEM((ABLK * CA,), jnp.int32),
        pltpu.VMEM((ABLK * CA,), jnp.int32),
        pltpu.VMEM((CA,), F32),
        pltpu.VMEM((16, 16), F32),
        pltpu.VMEM((NSL,), F32),
        pltpu.VMEM((NSL,), F32),
        pltpu.VMEM_SHARED((NS, NP_), F32),
        pltpu.SemaphoreType.DMA,
        pltpu.SemaphoreType.DMA,
    ],
  )


def _k_softmax(logits, dst, mpart,
               ex, dpart,
               m_full, den_t, blog, bdst, bex, tmpa, tmpb, den_sh, sem):
    cid = lax.axis_index("c")
    sid = lax.axis_index("s")
    wid = sid * NC + cid

    pltpu.sync_copy(mpart.at[0], m_full)
    pltpu.sync_copy(mpart.at[1], den_t)
    _vloop(m_full, NP_,
           lambda i: jnp.maximum(m_full[pl.ds(i * 16, 16)],
                                 den_t[pl.ds(i * 16, 16)]))
    zero = jnp.zeros((16,), F32)
    _vloop(den_t, NP_, lambda i: zero)

    ebase = wid * E_W_A

    def chunk(it, carry):
        base = ebase + it * CC
        pltpu.sync_copy(logits.at[pl.ds(base, CC)], blog)
        pltpu.sync_copy(dst.at[pl.ds(base, CC)], bdst)

        def grp(g, c2):
            l16 = blog[pl.ds(g * 16, 16)]
            d16 = bdst[pl.ds(g * 16, 16)]
            mv = plsc.load_gather(m_full, [d16])
            e16 = jnp.exp(l16 - mv)
            bex[pl.ds(g * 16, 16)] = e16
            _table_update(den_t, d16, e16, lambda a, b: a + b)
            return c2
        lax.fori_loop(0, CC // 16, grp, None)
        pltpu.sync_copy(bex, ex.at[pl.ds(base, CC)])
        return carry
    lax.fori_loop(0, E_W_A // CC, chunk, None)

    pltpu.sync_copy(den_t, den_sh.at[sid])
    plsc.subcore_barrier()
    nb = sid * NSL
    pltpu.sync_copy(den_sh.at[0, pl.ds(nb, NSL)], tmpa)
    for w in range(1, NS):
        pltpu.sync_copy(den_sh.at[w, pl.ds(nb, NSL)], tmpb)
        _vloop(tmpa, NSL,
               lambda i: tmpa[pl.ds(i * 16, 16)] + tmpb[pl.ds(i * 16, 16)])
    pltpu.sync_copy(tmpa, dpart.at[cid, pl.ds(nb, NSL)])


@functools.lru_cache(maxsize=None)
def _softmax_call():
  return pl.kernel(
    _k_softmax,
    out_type=(jax.ShapeDtypeStruct((E,), F32),
              jax.ShapeDtypeStruct((NC, NP_), F32)),
    mesh=_mesh(),
    compiler_params=pltpu.CompilerParams(needs_layout_passes=False),
    scratch_types=[
        pltpu.VMEM((NP_,), F32),
        pltpu.VMEM((NP_,), F32),
        pltpu.VMEM((CC,), F32),
        pltpu.VMEM((CC,), jnp.int32),
        pltpu.VMEM((CC,), F32),
        pltpu.VMEM((NSL,), F32),
        pltpu.VMEM((NSL,), F32),
        pltpu.VMEM_SHARED((NS, NP_), F32),
        pltpu.SemaphoreType.DMA,
    ],
  )


NCH_D = E_W_D // CD
DBLK = 50
NPD = 10112
NSD = NPD // NS


def _k_aggr(ex, srcr, dstr, dpart, xllo, xlhi, biasr,
            hlo, hhi,
            den_full, brow0, brow1, bsrc, bdst, bex, bdsts, bbias,
            fbuf, out_sh, sg0, sg1, ss0, ss1, *, use_silu):
    cid = lax.axis_index("c")
    sid = lax.axis_index("s")

    pltpu.sync_copy(dpart.at[pl.ds(0, NP_)], den_full)
    for k, sz in [(kk, 800) for kk in range(0, 9600, 800)] + [(9600, 640)]:
        pltpu.sync_copy(dpart.at[pl.ds(NP_ + k, sz)], bex.at[pl.ds(0, sz)])

        def comb(i, c2, k=k):
            den_full[pl.ds(k + i * 16, 16)] = (
                den_full[pl.ds(k + i * 16, 16)] + bex[pl.ds(i * 16, 16)])
            return c2
        lax.fori_loop(0, sz // 16, comb, None)
    pltpu.sync_copy(biasr.at[cid], bbias)

    ebase = sid * E_W_D

    zero = jnp.zeros((16,), F32)

    def zb(i, c2):
        r = i // 8
        cc = i % 8
        fbuf[r, pl.ds(cc * 16, 16)] = zero
        return c2
    lax.fori_loop(0, 8 * 8, zb, None)

    def core_half(xl_half, h_half):
        for b in range(NSD // 8):
            pltpu.sync_copy(fbuf, out_sh.at[pl.ds(sid * NSD + b * 8, 8)])
        plsc.subcore_barrier()

        brows = (brow0, brow1)
        sems = (sg0, sg1)
        ssem = (ss0, ss1)

        def start_g(p, j):
            pltpu.async_copy(xl_half.at[bsrc.at[pl.ds(j * CD, CD)]],
                             brows[p], sems[p])

        def wait_scatter(p):
            pltpu.make_async_copy(brows[p], out_sh.at[bdsts], ssem[p]).wait()

        def block(bb, c1):
            eblk = ebase + bb * (DBLK * CD)
            pltpu.sync_copy(srcr.at[pl.ds(eblk, DBLK * CD)], bsrc)
            pltpu.sync_copy(dstr.at[pl.ds(eblk, DBLK * CD)], bdst)
            pltpu.sync_copy(ex.at[pl.ds(eblk, DBLK * CD)], bex)
            start_g(0, 0)

            def step(j, c2):
                def proc(p):
                    cur = brows[p]

                    @pl.when((j > 0) | (bb > 0))
                    def _():
                        wait_scatter(1 - p)

                    @pl.when(j + 1 < DBLK)
                    def _():
                        start_g(1 - p, j + 1)
                    pltpu.make_async_copy(
                        xl_half.at[bsrc.at[pl.ds(0, CD)]], cur,
                        sems[p]).wait()

                    def grp(g, c3):
                        d16 = bdst[pl.ds(j * CD + g * 16, 16)]
                        bdsts[pl.ds(g * 16, 16)] = d16
                        e16 = bex[pl.ds(j * CD + g * 16, 16)]
                        dv = plsc.load_gather(den_full, [d16])
                        av16 = e16 / (dv + 1e-16)
                        for jj in range(16):
                            avj = jnp.take_along_axis(
                                av16, jnp.full((16,), jj, jnp.int32),
                                axis=0, mode="promise_in_bounds")
                            e2 = g * 16 + jj
                            for cc in range(8):
                                cur[e2, pl.ds(cc * 16, 16)] = (
                                    cur[e2, pl.ds(cc * 16, 16)] * avj)
                        return c3
                    lax.fori_loop(0, CD // 16, grp, None)
                    pltpu.async_copy(cur, out_sh.at[bdsts], ssem[p],
                                     add=True)

                @pl.when(j % 2 == 0)
                def _():
                    proc(0)

                @pl.when(j % 2 == 1)
                def _():
                    proc(1)
                return c2
            lax.fori_loop(0, DBLK, step, None)
            return c1
        lax.fori_loop(0, NCH_D // DBLK, block, None)
        wait_scatter((DBLK - 1) % 2)
        plsc.subcore_barrier()

        def finblk(b, c1):
            rbase = sid * NSD + b * 8
            pltpu.sync_copy(out_sh.at[pl.ds(rbase, 8)], fbuf)

            def fin(i, c2):
                r = i // 8
                cc = i % 8
                v = fbuf[r, pl.ds(cc * 16, 16)] + bbias[cc]
                if use_silu:
                    v = v / (1.0 + jnp.exp(-v))
                fbuf[r, pl.ds(cc * 16, 16)] = v
                return c2
            lax.fori_loop(0, 8 * 8, fin, None)
            pltpu.sync_copy(fbuf, h_half.at[pl.ds(rbase, 8)])
            return c1
        lax.fori_loop(0, NSD // 8, finblk, None)

    @pl.when(cid == 0)
    def _():
        core_half(xllo, hlo)

    @pl.when(cid == 1)
    def _():
        core_half(xlhi, hhi)


@functools.lru_cache(maxsize=None)
def _make_aggr(use_silu):
    return pl.kernel(
        functools.partial(_k_aggr, use_silu=use_silu),
        out_type=(jax.ShapeDtypeStruct((NPD, HH), F32),
                  jax.ShapeDtypeStruct((NPD, HH), F32)),
        mesh=_mesh(),
        compiler_params=pltpu.CompilerParams(needs_layout_passes=False),
        scratch_types=[
            pltpu.VMEM((NP_,), F32),
            pltpu.VMEM((CD, HH), F32),
            pltpu.VMEM((CD, HH), F32),
            pltpu.VMEM((DBLK * CD,), jnp.int32),
            pltpu.VMEM((DBLK * CD,), jnp.int32),
            pltpu.VMEM((DBLK * CD,), F32),
            pltpu.VMEM((CD,), jnp.int32),
            pltpu.VMEM((8, 16), F32),
            pltpu.VMEM((8, HH), F32),
            pltpu.VMEM_SHARED((NPD, HH), F32),
            pltpu.SemaphoreType.DMA,
            pltpu.SemaphoreType.DMA,
            pltpu.SemaphoreType.DMA,
            pltpu.SemaphoreType.DMA,
        ],
    )


def _pack_pair(lo, hi):
    bl = lax.bitcast_convert_type(lo, jnp.uint32)
    bh = lax.bitcast_convert_type(hi, jnp.uint32)
    rl = (bl + 0x7FFF + ((bl >> 16) & 1)) >> 16
    rh = ((bh + 0x7FFF + ((bh >> 16) & 1)) >> 16) << 16
    return lax.bitcast_convert_type(rl | rh, jnp.int32)


def _pack_words(xf):
    return _pack_pair(xf[:, :HH], xf[:, HH:])


def _proj1_body(x_ref, wl_ref, bl_ref, wr_ref, br_ref,
                xllo, xlhi, xlb, xrb):
    xv = x_ref[...]
    xl = jnp.dot(xv, wl_ref[...], preferred_element_type=F32,
                 precision=lax.Precision.HIGHEST) + bl_ref[...]
    xr = jnp.dot(xv, wr_ref[...], preferred_element_type=F32,
                 precision=lax.Precision.HIGHEST) + br_ref[...]
    xllo[...] = xl[:, :HH]
    xlhi[...] = xl[:, HH:]
    xlb[...] = _pack_words(xl)
    xrb[...] = _pack_words(xr)


_RB = 1000

_proj1 = pl.pallas_call(
    _proj1_body,
    grid=(N // _RB,),
    in_specs=[
        pl.BlockSpec((_RB, 128), lambda i: (i, 0)),
        pl.BlockSpec((128, DH), lambda i: (0, 0)),
        pl.BlockSpec((DH,), lambda i: (0,)),
        pl.BlockSpec((128, DH), lambda i: (0, 0)),
        pl.BlockSpec((DH,), lambda i: (0,)),
    ],
    out_specs=[pl.BlockSpec((_RB, HH), lambda i: (i, 0))] * 4,
    out_shape=[jax.ShapeDtypeStruct((N, HH), F32)] * 2
    + [jax.ShapeDtypeStruct((N, HH), jnp.int32)] * 2,
)


def _proj2_body(xlo_ref, xhi_ref, wl_ref, bl_ref, wr_ref, br_ref,
                xllo, xlhi, xlb, xrb):
    lo = xlo_ref[...]
    hi = xhi_ref[...]
    xl = (jnp.dot(lo, wl_ref[:HH, :], preferred_element_type=F32,
                  precision=lax.Precision.HIGHEST)
          + jnp.dot(hi, wl_ref[HH:, :], preferred_element_type=F32,
                    precision=lax.Precision.HIGHEST) + bl_ref[...])
    xr = (jnp.dot(lo, wr_ref[:HH, :], preferred_element_type=F32,
                  precision=lax.Precision.HIGHEST)
          + jnp.dot(hi, wr_ref[HH:, :], preferred_element_type=F32,
                    precision=lax.Precision.HIGHEST) + br_ref[...])
    xllo[...] = xl[:, :HH]
    xlhi[...] = xl[:, HH:]
    xlb[...] = _pack_words(xl)
    xrb[...] = _pack_words(xr)


_proj2 = pl.pallas_call(
    _proj2_body,
    grid=(N // _RB,),
    in_specs=[
        pl.BlockSpec((_RB, HH), lambda i: (i, 0)),
        pl.BlockSpec((_RB, HH), lambda i: (i, 0)),
        pl.BlockSpec((DH, DH), lambda i: (0, 0)),
        pl.BlockSpec((DH,), lambda i: (0,)),
        pl.BlockSpec((DH, DH), lambda i: (0, 0)),
        pl.BlockSpec((DH,), lambda i: (0,)),
    ],
    out_specs=[pl.BlockSpec((_RB, HH), lambda i: (i, 0))] * 4,
    out_shape=[jax.ShapeDtypeStruct((N, HH), F32)] * 2
    + [jax.ShapeDtypeStruct((N, HH), jnp.int32)] * 2,
)


def _ew_body(ea_ref, we_ref, out_ref):
    out_ref[...] = _pack_words(
        jnp.dot(ea_ref[...], we_ref[...], preferred_element_type=F32,
                precision=lax.Precision.HIGHEST))


_EB = 4000

_ew_call = pl.pallas_call(
    _ew_body,
    grid=(E // _EB,),
    in_specs=[
        pl.BlockSpec((_EB, D_E), lambda i: (i, 0)),
        pl.BlockSpec((D_E, DH), lambda i: (0, 0)),
    ],
    out_specs=pl.BlockSpec((_EB, HH), lambda i: (i, 0)),
    out_shape=jax.ShapeDtypeStruct((E, HH), jnp.int32),
)


def _att_arrange(att):
    return att.reshape(2, 8, 16).transpose(1, 0, 2).reshape(16, 16)


def _gat_layer(xllo, xlhi, xlb, xrb, ewb, src, dst, att, bias, use_silu):
    att_r = _att_arrange(att)
    bias_r = jnp.stack([bias[:HH].reshape(8, 16), bias[HH:].reshape(8, 16)])
    logits, mpart = _attn_call()(xlb, xrb, ewb, src, dst, att_r)
    ex, dpart = _softmax_call()(logits, dst, mpart)
    call = _make_aggr(use_silu)
    hlo, hhi = call(ex, src, dst, dpart.reshape(2 * NP_), xllo, xlhi, bias_r)
    return hlo[:N], hhi[:N]


def kernel(x, edge_index, edge_attr,
           Wl0, bl0, Wr0, br0, We0, att0, bias0,
           Wl1, bl1, Wr1, br1, We1, att1, bias1):
    src = edge_index[0]
    dst = edge_index[1]
    ewb0 = _ew_call(edge_attr, We0)
    ewb1 = _ew_call(edge_attr, We1)
    xllo, xlhi, xlb, xrb = _proj1(x, Wl0, bl0, Wr0, br0)
    h0lo, h0hi = _gat_layer(xllo, xlhi, xlb, xrb, ewb0, src, dst,
                            att0, bias0, True)
    xllo1, xlhi1, xlb1, xrb1 = _proj2(h0lo, h0hi, Wl1, bl1, Wr1, br1)
    h1lo, h1hi = _gat_layer(xllo1, xlhi1, xlb1, xrb1, ewb1, src, dst,
                            att1, bias1, False)
    return jnp.concatenate([h1lo, h1hi], axis=1)
